# baseline (speedup 1.0000x reference)
import functools

import jax
import jax.numpy as jnp
from jax import lax
from jax.experimental import pallas as pl
from jax.experimental.pallas import tpu as pltpu
from jax.experimental.pallas import tpu_sc as plsc

N = 1024
E = 4096
LATENT = 128
MP = 3
DT = 0.01
LN_EPS = 1e-6

_f32 = jnp.float32
_EB = 1024
_NEB = E // _EB

_NC = 2
_NS = 16
_NW = _NC * _NS
_EPW = E // _NW
_RPT = N // _NS


def _mm(a, b):
    return lax.dot_general(a, b, (((1,), (0,)), ((), ())),
                           preferred_element_type=_f32)


def _mmT(a, b):
    return lax.dot_general(a, b, (((1,), (1,)), ((), ())),
                           preferred_element_type=_f32)


def _ln(z):
    mu = jnp.mean(z, axis=-1, keepdims=True)
    zc = z - mu
    var = jnp.mean(zc * zc, axis=-1, keepdims=True)
    sig = jnp.sqrt(var + LN_EPS)
    return zc / sig, sig


def _ln_bwd(gy, y, sig):
    return (gy - jnp.mean(gy, axis=-1, keepdims=True)
            - y * jnp.mean(gy * y, axis=-1, keepdims=True)) / sig


def _sc_mesh():
    return plsc.VectorSubcoreMesh(core_axis_name="c", subcore_axis_name="s")


def _wid():
    return lax.axis_index("s") * _NC + lax.axis_index("c")


def _gather2_body(tbl, idxa, idxb, outa, outb,
                  ia_v, ib_v, ra_v, rb_v, sia, sib, sga, sgb, swa, swb):
    base = _wid() * _EPW
    cia = pltpu.async_copy(idxa.at[pl.ds(base, _EPW)], ia_v, sia)
    cib = pltpu.async_copy(idxb.at[pl.ds(base, _EPW)], ib_v, sib)
    cia.wait()
    cga = pltpu.async_copy(tbl.at[ia_v], ra_v, sga)
    cib.wait()
    cgb = pltpu.async_copy(tbl.at[ib_v], rb_v, sgb)
    cga.wait()
    cwa = pltpu.async_copy(ra_v, outa.at[pl.ds(base, _EPW)], swa)
    cgb.wait()
    cwb = pltpu.async_copy(rb_v, outb.at[pl.ds(base, _EPW)], swb)
    cwa.wait()
    cwb.wait()


@functools.cache
def _mk_gather2():
    return pl.kernel(
        _gather2_body,
        out_type=[jax.ShapeDtypeStruct((E, LATENT), _f32),
                  jax.ShapeDtypeStruct((E, LATENT), _f32)],
        mesh=_sc_mesh(),
        scratch_types=[pltpu.VMEM((_EPW,), jnp.int32),
                       pltpu.VMEM((_EPW,), jnp.int32),
                       pltpu.VMEM((_EPW, LATENT), _f32),
                       pltpu.VMEM((_EPW, LATENT), _f32)]
        + [pltpu.SemaphoreType.DMA] * 6,
    )


def _gather2(tbl, ia, ib):
    return _mk_gather2()(tbl, ia, ib)


def _gather1_body(tbl, idxa, outa, idx_v, rows_v, sem):
    base = _wid() * _EPW
    pltpu.sync_copy(idxa.at[pl.ds(base, _EPW)], idx_v)
    pltpu.async_copy(tbl.at[idx_v], rows_v, sem).wait()
    pltpu.sync_copy(rows_v, outa.at[pl.ds(base, _EPW)])


def _halfrow(i):
    return (i, i)


@functools.cache
def _mk_gather1():
    return pl.kernel(
        _gather1_body,
        out_type=jax.ShapeDtypeStruct((E, LATENT), _f32),
        mesh=_sc_mesh(),
        scratch_types=[pltpu.VMEM((_EPW,), jnp.int32),
                       pltpu.VMEM((_EPW, LATENT), _f32),
                       pltpu.SemaphoreType.DMA],
    )


def _gather1(tbl, ia):
    return _mk_gather1()(tbl, ia)


def _scat1_body(src, idx, init, out, acc, idx_v, rows_v, s0, s1, s2):
    c = lax.axis_index("c")
    s = lax.axis_index("s")
    base = _wid() * _EPW
    rbase = s * _RPT
    ci = pltpu.async_copy(init.at[c, pl.ds(rbase, _RPT)],
                          acc.at[pl.ds(rbase, _RPT)], s0)
    cx = pltpu.async_copy(idx.at[pl.ds(base, _EPW)], idx_v, s1)
    cs = pltpu.async_copy(src.at[pl.ds(base, _EPW)], rows_v, s2)
    ci.wait()
    cx.wait()
    cs.wait()
    plsc.subcore_barrier()
    pltpu.sync_copy(rows_v, acc.at[idx_v], add=True)
    plsc.subcore_barrier()
    pltpu.sync_copy(acc.at[pl.ds(rbase, _RPT)], out.at[c, pl.ds(rbase, _RPT)])


@functools.cache
def _mk_scat1():
    return pl.kernel(
        _scat1_body,
        out_type=jax.ShapeDtypeStruct((_NC, N, LATENT), _f32),
        mesh=_sc_mesh(),
        scratch_types=[pltpu.VMEM_SHARED((N, LATENT), _f32),
                       pltpu.VMEM((_EPW,), jnp.int32),
                       pltpu.VMEM((_EPW, LATENT), _f32)]
        + [pltpu.SemaphoreType.DMA] * 3,
    )


def _scat1(src_, idx, init):
    return _mk_scat1()(src_, idx, init)


def _scat2_body(srca, idxa, srcb, idxb, init, out, acc,
                ia_v, ib_v, ra_v, rb_v, s0, s1, s2, s3, s4):
    c = lax.axis_index("c")
    s = lax.axis_index("s")
    base = _wid() * _EPW
    rbase = s * _RPT
    ci = pltpu.async_copy(init.at[c, pl.ds(rbase, _RPT)],
                          acc.at[pl.ds(rbase, _RPT)], s0)
    ca = pltpu.async_copy(idxa.at[pl.ds(base, _EPW)], ia_v, s1)
    cb = pltpu.async_copy(idxb.at[pl.ds(base, _EPW)], ib_v, s2)
    cra = pltpu.async_copy(srca.at[pl.ds(base, _EPW)], ra_v, s3)
    crb = pltpu.async_copy(srcb.at[pl.ds(base, _EPW)], rb_v, s4)
    ci.wait()
    ca.wait()
    cb.wait()
    cra.wait()
    crb.wait()
    plsc.subcore_barrier()
    pltpu.sync_copy(ra_v, acc.at[ia_v], add=True)
    pltpu.sync_copy(rb_v, acc.at[ib_v], add=True)
    plsc.subcore_barrier()
    pltpu.sync_copy(acc.at[pl.ds(rbase, _RPT)], out.at[c, pl.ds(rbase, _RPT)])


@functools.cache
def _mk_scat2():
    return pl.kernel(
        _scat2_body,
        out_type=jax.ShapeDtypeStruct((_NC, N, LATENT), _f32),
        mesh=_sc_mesh(),
        scratch_types=[pltpu.VMEM_SHARED((N, LATENT), _f32),
                       pltpu.VMEM((_EPW,), jnp.int32),
                       pltpu.VMEM((_EPW,), jnp.int32),
                       pltpu.VMEM((_EPW, LATENT), _f32),
                       pltpu.VMEM((_EPW, LATENT), _f32)]
        + [pltpu.SemaphoreType.DMA] * 5,
    )


def _scat2(sa, ia, sb, ib, init):
    return _mk_scat2()(sa, ia, sb, ib, init)


def _eblk(i):
    return (i, 0)


def _full(i):
    return (0, 0)


def _espec(w=LATENT):
    return pl.BlockSpec((_EB, w), _eblk)


def _wspec(shape):
    return pl.BlockSpec(shape, _full)


def _enc_body(state, t2, nodes, w10, b10, W11, b11, w20, b20, W21, b21,
              Wn0, bn0, Wn1, bn1, Eh0, V0):
    st = state[...]
    a1 = jnp.maximum(st * w10[...] + b10[...], 0.0)
    e1, _ = _ln(_mm(a1, W11[...]) + b11[...])
    a2 = jnp.maximum(st * w20[...] + b20[...], 0.0)
    e2, _ = _ln(_mm(a2, W21[...]) + b21[...])
    Eh0[...] = e1 + t2[...] * (e2 - e1)

    @pl.when(pl.program_id(0) == 0)
    def _():
        a = jnp.maximum(_mm(nodes[...], Wn0[...]) + bn0[...], 0.0)
        V0[...], _ = _ln(_mm(a, Wn1[...]) + bn1[...])


def _run_enc(state_col, t2_col, nodes, ew, nw):
    return pl.pallas_call(
        _enc_body,
        grid=(_NEB,),
        in_specs=[pl.BlockSpec((_EB, 1), _eblk), pl.BlockSpec((_EB, 1), _eblk),
                  _wspec(nodes.shape)]
        + [_wspec(w.shape) for w in ew] + [_wspec(w.shape) for w in nw],
        out_specs=[_espec(), pl.BlockSpec((N, LATENT), _full)],
        out_shape=[jax.ShapeDtypeStruct((E, LATENT), _f32),
                   jax.ShapeDtypeStruct((N, LATENT), _f32)],
    )(state_col, t2_col, nodes, *ew, *nw)


def _edge_fwd_body(Eh, Vs, Vr, P0, p0, P1, p1, Ehn, h_o, m_o, sig_o):
    c = jnp.concatenate([Eh[...], Vs[...], Vr[...]], axis=-1)
    h = jnp.maximum(_mm(c, P0[...]) + p0[...], 0.0)
    m, sig = _ln(_mm(h, P1[...]) + p1[...])
    Ehn[...] = Eh[...] + m
    h_o[...] = h
    m_o[...] = m
    sig_o[...] = sig


def _run_edge_fwd(Eh, Vs, Vr, pw):
    return pl.pallas_call(
        _edge_fwd_body,
        grid=(_NEB,),
        in_specs=[_espec(), _espec(), _espec()]
        + [_wspec(w.shape) for w in pw],
        out_specs=[_espec(), _espec(), _espec(), pl.BlockSpec((_EB, 1), _eblk)],
        out_shape=[jax.ShapeDtypeStruct((E, LATENT), _f32)] * 3
        + [jax.ShapeDtypeStruct((E, 1), _f32)],
    )(Eh, Vs, Vr, *pw)


def _node_fwd_body(V, aggp, Q0, q0, Q1, q1, Vn, k_o, u_o, sig_o):
    agg = aggp[0] + aggp[1]
    d = jnp.concatenate([V[...], agg], axis=-1)
    k = jnp.maximum(_mm(d, Q0[...]) + q0[...], 0.0)
    u, sig = _ln(_mm(k, Q1[...]) + q1[...])
    Vn[...] = V[...] + u
    k_o[...] = k
    u_o[...] = u
    sig_o[...] = sig


def _run_node_fwd(V, aggp, qw):
    return pl.pallas_call(
        _node_fwd_body,
        out_shape=[jax.ShapeDtypeStruct((N, LATENT), _f32)] * 3
        + [jax.ShapeDtypeStruct((N, 1), _f32)],
    )(V, aggp, *qw)


def _mid_body(Eh, Vs, Vr, t2, P0, p0, P1, p1,
              A0, a0, A1, a1, a2r, B0, b0, B1, b1, b2r,
              gE_o, gcs_o, gcr_o):
    c = jnp.concatenate([Eh[...], Vs[...], Vr[...]], axis=-1)
    h = jnp.maximum(_mm(c, P0[...]) + p0[...], 0.0)
    m, sig = _ln(_mm(h, P1[...]) + p1[...])
    Eh3 = Eh[...] + m
    t2v = t2[...]
    y11 = jnp.maximum(_mm(Eh3, A0[...]) + a0[...], 0.0)
    y12 = jnp.maximum(_mm(y11, A1[...]) + a1[...], 0.0)
    y21 = jnp.maximum(_mm(Eh3, B0[...]) + b0[...], 0.0)
    y22 = jnp.maximum(_mm(y21, B1[...]) + b1[...], 0.0)
    g12 = (1.0 - t2v) * a2r[...] * (y12 > 0)
    g11 = _mmT(g12, A1[...]) * (y11 > 0)
    gE = _mmT(g11, A0[...])
    g22 = t2v * b2r[...] * (y22 > 0)
    g21 = _mmT(g22, B1[...]) * (y21 > 0)
    gE = gE + _mmT(g21, B0[...])
    gz = _ln_bwd(gE, m, sig)
    gh = _mmT(gz, P1[...]) * (h > 0)
    gc = _mmT(gh, P0[...])
    gE_o[...] = gE + gc[:, :LATENT]
    gcs_o[...] = gc[:, LATENT:2 * LATENT]
    gcr_o[...] = gc[:, 2 * LATENT:]


def _run_mid(Eh, Vs, Vr, t2_col, pw, dw):
    return pl.pallas_call(
        _mid_body,
        grid=(_NEB,),
        in_specs=[_espec(), _espec(), _espec(), pl.BlockSpec((_EB, 1), _eblk)]
        + [_wspec(w.shape) for w in pw + dw],
        out_specs=[_espec(), _espec(), _espec()],
        out_shape=[jax.ShapeDtypeStruct((E, LATENT), _f32)] * 3,
    )(Eh, Vs, Vr, t2_col, *pw, *dw)


def _node_bwd_body(gVp, u, sigw, k, Q0, Q1, gVmid_o, gagg_o):
    gV = gVp[0] + gVp[1]
    gw = _ln_bwd(gV, u[...], sigw[...])
    gk = _mmT(gw, Q1[...]) * (k[...] > 0)
    gd = _mmT(gk, Q0[...])
    gVmid_o[...] = gV + gd[:, :LATENT]
    gagg_o[...] = gd[:, LATENT:]


def _run_node_bwd(gVp, u, sigw, k, Q0, Q1):
    return pl.pallas_call(
        _node_bwd_body,
        out_shape=[jax.ShapeDtypeStruct((N, LATENT), _f32)] * 2,
    )(gVp, u, sigw, k, Q0, Q1)


def _edge_bwd_body(gE_in, garr, m, sig, h, P0, P1, gE_o, gcs_o, gcr_o):
    gE = gE_in[...] + garr[...]
    gz = _ln_bwd(gE, m[...], sig[...])
    gh = _mmT(gz, P1[...]) * (h[...] > 0)
    gc = _mmT(gh, P0[...])
    gE_o[...] = gE + gc[:, :LATENT]
    gcs_o[...] = gc[:, LATENT:2 * LATENT]
    gcr_o[...] = gc[:, 2 * LATENT:]


def _run_edge_bwd(gE_in, garr, m, sig, h, P0, P1):
    return pl.pallas_call(
        _edge_bwd_body,
        grid=(_NEB,),
        in_specs=[_espec(), _espec(), _espec(), pl.BlockSpec((_EB, 1), _eblk),
                  _espec(), _wspec(P0.shape), _wspec(P1.shape)],
        out_specs=[_espec(), _espec(), _espec()],
        out_shape=[jax.ShapeDtypeStruct((E, LATENT), _f32)] * 3,
    )(gE_in, garr, m, sig, h, P0, P1)


def _edge_bwd0_enc_body(gE_in, garr, m, sig, h, P0, P1,
                        state, t2, w10, b10, W11, b11, w20, b20, W21, b21,
                        dH_o):
    gE = gE_in[...] + garr[...]
    gz = _ln_bwd(gE, m[...], sig[...])
    gh = _mmT(gz, P1[...]) * (h[...] > 0)
    gc = _mmT(gh, P0[...])
    gE0 = gE + gc[:, :LATENT]
    st = state[...]
    t2v = t2[...]
    a1 = jnp.maximum(st * w10[...] + b10[...], 0.0)
    e1, sig1 = _ln(_mm(a1, W11[...]) + b11[...])
    a2 = jnp.maximum(st * w20[...] + b20[...], 0.0)
    e2, sig2 = _ln(_mm(a2, W21[...]) + b21[...])
    gz1 = _ln_bwd((1.0 - t2v) * gE0, e1, sig1)
    ga1 = _mmT(gz1, W11[...]) * (a1 > 0)
    gs1 = jnp.sum(ga1 * w10[...], axis=-1, keepdims=True)
    gz2 = _ln_bwd(t2v * gE0, e2, sig2)
    ga2 = _mmT(gz2, W21[...]) * (a2 > 0)
    gs2 = jnp.sum(ga2 * w20[...], axis=-1, keepdims=True)
    dH_o[...] = gs1 + gs2


def _run_edge_bwd0_enc(gE_in, garr, m, sig, h, P0, P1, state_col, t2_col, ew):
    return pl.pallas_call(
        _edge_bwd0_enc_body,
        grid=(_NEB,),
        in_specs=[_espec(), _espec(), _espec(), pl.BlockSpec((_EB, 1), _eblk),
                  _espec(), _wspec(P0.shape), _wspec(P1.shape),
                  pl.BlockSpec((_EB, 1), _eblk), pl.BlockSpec((_EB, 1), _eblk)]
        + [_wspec(w.shape) for w in ew],
        out_specs=pl.BlockSpec((_EB, 1), _eblk),
        out_shape=jax.ShapeDtypeStruct((E, 1), _f32),
    )(gE_in, garr, m, sig, h, P0, P1, state_col, t2_col, *ew)


_JBLK = 512


def _jg_body(J_ref, g_ref, dHr_ref, dHc_ref, ctrl_ref, state_ref,
             out_ref, acc_ref):
    i = pl.program_id(0)

    @pl.when(i == 0)
    def _():
        acc_ref[...] = jnp.zeros_like(acc_ref)

    rows = lax.broadcasted_iota(jnp.int32, (_JBLK, E), 0) + i * _JBLK
    cols = lax.broadcasted_iota(jnp.int32, (_JBLK, E), 1)
    Jm = J_ref[...] * (cols >= rows).astype(_f32)
    y_rows = jnp.sum(Jm * dHr_ref[...], axis=1)
    cc = jnp.sum(Jm * dHc_ref[...], axis=0)
    acc_ref[0, :] = acc_ref[0, :] + cc
    gc = jnp.sum(g_ref[...] * ctrl_ref[...], axis=1)
    y_cols = acc_ref[0, pl.ds(i * _JBLK, _JBLK)]
    out_ref[:, 0] = state_ref[:, 0] + DT * (y_rows - y_cols + gc)


def _run_jg(J, g, dH_row, dH_col, ctrl_row, state_col):
    return pl.pallas_call(
        _jg_body,
        grid=(E // _JBLK,),
        in_specs=[
            pl.BlockSpec((_JBLK, E), _eblk),
            pl.BlockSpec((_JBLK, E), _eblk),
            pl.BlockSpec((1, E), _full),
            pl.BlockSpec((_JBLK, 1), _eblk),
            pl.BlockSpec((1, E), _full),
            pl.BlockSpec((_JBLK, 1), _eblk),
        ],
        out_specs=pl.BlockSpec((_JBLK, 1), _eblk),
        out_shape=jax.ShapeDtypeStruct((E, 1), _f32),
        scratch_shapes=[pltpu.VMEM((1, E), _f32)],
    )(J, g, dH_row, dH_col, ctrl_row, state_col)


def kernel(nodes, state, senders, receivers, type2_mask, control, J, g, params):
    state_col = state[:, None]
    send = senders.astype(jnp.int32)
    recv = receivers.astype(jnp.int32)
    t2_col = type2_mask[:, None].astype(_f32)

    def row(v):
        return v.reshape(1, -1)

    pe = params
    nw = (pe["enc_node"][0][0], row(pe["enc_node"][0][1]),
          pe["enc_node"][1][0], row(pe["enc_node"][1][1]))
    ew = (pe["enc_e1"][0][0], row(pe["enc_e1"][0][1]),
          pe["enc_e1"][1][0], row(pe["enc_e1"][1][1]),
          pe["enc_e2"][0][0], row(pe["enc_e2"][0][1]),
          pe["enc_e2"][1][0], row(pe["enc_e2"][1][1]))
    pw = (pe["proc_edge"][0][0], row(pe["proc_edge"][0][1]),
          pe["proc_edge"][1][0], row(pe["proc_edge"][1][1]))
    qw = (pe["proc_node"][0][0], row(pe["proc_node"][0][1]),
          pe["proc_node"][1][0], row(pe["proc_node"][1][1]))
    dw = (pe["dec_e1"][0][0], row(pe["dec_e1"][0][1]),
          pe["dec_e1"][1][0], row(pe["dec_e1"][1][1]),
          row(pe["dec_e1"][2][0][:, 0]),
          pe["dec_e2"][0][0], row(pe["dec_e2"][0][1]),
          pe["dec_e2"][1][0], row(pe["dec_e2"][1][1]),
          row(pe["dec_e2"][2][0][:, 0]))
    Q0, Q1 = qw[0], qw[2]
    P0, P1 = pw[0], pw[2]

    zinit = jnp.zeros((_NC, N, LATENT), _f32)

    Eh0, V0 = _run_enc(state_col, t2_col, nodes, ew, nw)

    Vs0, Vr0 = _gather2(V0, send, recv)
    Eh1, h0, m0, sz0 = _run_edge_fwd(Eh0, Vs0, Vr0, pw)
    aggp0 = _scat1(Eh1, recv, zinit)
    V1, k0, u0, sw0 = _run_node_fwd(V0, aggp0, qw)

    Vs1, Vr1 = _gather2(V1, send, recv)
    Eh2, h1, m1, sz1 = _run_edge_fwd(Eh1, Vs1, Vr1, pw)
    aggp1 = _scat1(Eh2, recv, zinit)
    V2, k1, u1, sw1 = _run_node_fwd(V1, aggp1, qw)

    Vs2, Vr2 = _gather2(V2, send, recv)
    gE2, gcs2, gcr2 = _run_mid(Eh2, Vs2, Vr2, t2_col, pw, dw)

    gVp2 = _scat2(gcs2, send, gcr2, recv, zinit)
    gVmid1, gagg1 = _run_node_bwd(gVp2, u1, sw1, k1, Q0, Q1)
    garr1 = _gather1(gagg1, recv)
    gE1, gcs1, gcr1 = _run_edge_bwd(gE2, garr1, m1, sz1, h1, P0, P1)

    init1 = jnp.concatenate([gVmid1[None], jnp.zeros((1, N, LATENT), _f32)], 0)
    gVp1 = _scat2(gcs1, send, gcr1, recv, init1)
    _, gagg0 = _run_node_bwd(gVp1, u0, sw0, k0, Q0, Q1)
    garr0 = _gather1(gagg0, recv)
    dH_col = _run_edge_bwd0_enc(gE1, garr0, m0, sz0, h0, P0, P1,
                                state_col, t2_col, ew)

    dH_row = dH_col.reshape(1, E)
    next_col = _run_jg(J, g, dH_row, dH_col, control.reshape(1, E), state_col)
    return next_col[:, 0]


# speedup vs baseline: 1.0391x; 1.0391x over previous
import functools

import jax
import jax.numpy as jnp
from jax import lax
from jax.experimental import pallas as pl
from jax.experimental.pallas import tpu as pltpu
from jax.experimental.pallas import tpu_sc as plsc

N = 1024
E = 4096
LATENT = 128
MP = 3
DT = 0.01
LN_EPS = 1e-6

_f32 = jnp.float32
_EB = 2048
_NEB = E // _EB

_NC = 2
_NS = 16
_NW = _NC * _NS
_EPW = E // _NW
_RPT = N // _NS


def _mm(a, b):
    return lax.dot_general(a, b, (((1,), (0,)), ((), ())),
                           preferred_element_type=_f32)


def _mmT(a, b):
    return lax.dot_general(a, b, (((1,), (1,)), ((), ())),
                           preferred_element_type=_f32)


def _ln(z):
    mu = jnp.mean(z, axis=-1, keepdims=True)
    zc = z - mu
    var = jnp.mean(zc * zc, axis=-1, keepdims=True)
    sig = jnp.sqrt(var + LN_EPS)
    return zc / sig, sig


def _ln_bwd(gy, y, sig):
    return (gy - jnp.mean(gy, axis=-1, keepdims=True)
            - y * jnp.mean(gy * y, axis=-1, keepdims=True)) / sig


def _sc_mesh():
    return plsc.VectorSubcoreMesh(core_axis_name="c", subcore_axis_name="s")


def _wid():
    return lax.axis_index("s") * _NC + lax.axis_index("c")


def _gather2_body(tbl, idxa, idxb, outa, outb,
                  ia_v, ib_v, ra_v, rb_v, sia, sib, sga, sgb, swa, swb):
    base = _wid() * _EPW
    cia = pltpu.async_copy(idxa.at[pl.ds(base, _EPW)], ia_v, sia)
    cib = pltpu.async_copy(idxb.at[pl.ds(base, _EPW)], ib_v, sib)
    cia.wait()
    cga = pltpu.async_copy(tbl.at[ia_v], ra_v, sga)
    cib.wait()
    cgb = pltpu.async_copy(tbl.at[ib_v], rb_v, sgb)
    cga.wait()
    cwa = pltpu.async_copy(ra_v, outa.at[pl.ds(base, _EPW)], swa)
    cgb.wait()
    cwb = pltpu.async_copy(rb_v, outb.at[pl.ds(base, _EPW)], swb)
    cwa.wait()
    cwb.wait()


@functools.cache
def _mk_gather2():
    return pl.kernel(
        _gather2_body,
        out_type=[jax.ShapeDtypeStruct((E, LATENT), _f32),
                  jax.ShapeDtypeStruct((E, LATENT), _f32)],
        mesh=_sc_mesh(),
        scratch_types=[pltpu.VMEM((_EPW,), jnp.int32),
                       pltpu.VMEM((_EPW,), jnp.int32),
                       pltpu.VMEM((_EPW, LATENT), _f32),
                       pltpu.VMEM((_EPW, LATENT), _f32)]
        + [pltpu.SemaphoreType.DMA] * 6,
    )


def _gather2(tbl, ia, ib):
    return _mk_gather2()(tbl, ia, ib)


def _gather1_body(tbl, idxa, outa, idx_v, rows_v, sem):
    base = _wid() * _EPW
    pltpu.sync_copy(idxa.at[pl.ds(base, _EPW)], idx_v)
    pltpu.async_copy(tbl.at[idx_v], rows_v, sem).wait()
    pltpu.sync_copy(rows_v, outa.at[pl.ds(base, _EPW)])


def _halfrow(i):
    return (i, i)


@functools.cache
def _mk_gather1():
    return pl.kernel(
        _gather1_body,
        out_type=jax.ShapeDtypeStruct((E, LATENT), _f32),
        mesh=_sc_mesh(),
        scratch_types=[pltpu.VMEM((_EPW,), jnp.int32),
                       pltpu.VMEM((_EPW, LATENT), _f32),
                       pltpu.SemaphoreType.DMA],
    )


def _gather1(tbl, ia):
    return _mk_gather1()(tbl, ia)


def _scat1_body(src, idx, init, out, acc, idx_v, rows_v, s0, s1, s2):
    c = lax.axis_index("c")
    s = lax.axis_index("s")
    base = _wid() * _EPW
    rbase = s * _RPT
    ci = pltpu.async_copy(init.at[c, pl.ds(rbase, _RPT)],
                          acc.at[pl.ds(rbase, _RPT)], s0)
    cx = pltpu.async_copy(idx.at[pl.ds(base, _EPW)], idx_v, s1)
    cs = pltpu.async_copy(src.at[pl.ds(base, _EPW)], rows_v, s2)
    ci.wait()
    cx.wait()
    cs.wait()
    plsc.subcore_barrier()
    pltpu.sync_copy(rows_v, acc.at[idx_v], add=True)
    plsc.subcore_barrier()
    pltpu.sync_copy(acc.at[pl.ds(rbase, _RPT)], out.at[c, pl.ds(rbase, _RPT)])


@functools.cache
def _mk_scat1():
    return pl.kernel(
        _scat1_body,
        out_type=jax.ShapeDtypeStruct((_NC, N, LATENT), _f32),
        mesh=_sc_mesh(),
        scratch_types=[pltpu.VMEM_SHARED((N, LATENT), _f32),
                       pltpu.VMEM((_EPW,), jnp.int32),
                       pltpu.VMEM((_EPW, LATENT), _f32)]
        + [pltpu.SemaphoreType.DMA] * 3,
    )


def _scat1(src_, idx, init):
    return _mk_scat1()(src_, idx, init)


def _scat2_body(srca, idxa, srcb, idxb, init, out, acc,
                ia_v, ib_v, ra_v, rb_v, s0, s1, s2, s3, s4):
    c = lax.axis_index("c")
    s = lax.axis_index("s")
    base = _wid() * _EPW
    rbase = s * _RPT
    ci = pltpu.async_copy(init.at[c, pl.ds(rbase, _RPT)],
                          acc.at[pl.ds(rbase, _RPT)], s0)
    ca = pltpu.async_copy(idxa.at[pl.ds(base, _EPW)], ia_v, s1)
    cb = pltpu.async_copy(idxb.at[pl.ds(base, _EPW)], ib_v, s2)
    cra = pltpu.async_copy(srca.at[pl.ds(base, _EPW)], ra_v, s3)
    crb = pltpu.async_copy(srcb.at[pl.ds(base, _EPW)], rb_v, s4)
    ci.wait()
    ca.wait()
    cb.wait()
    cra.wait()
    crb.wait()
    plsc.subcore_barrier()
    pltpu.sync_copy(ra_v, acc.at[ia_v], add=True)
    pltpu.sync_copy(rb_v, acc.at[ib_v], add=True)
    plsc.subcore_barrier()
    pltpu.sync_copy(acc.at[pl.ds(rbase, _RPT)], out.at[c, pl.ds(rbase, _RPT)])


@functools.cache
def _mk_scat2():
    return pl.kernel(
        _scat2_body,
        out_type=jax.ShapeDtypeStruct((_NC, N, LATENT), _f32),
        mesh=_sc_mesh(),
        scratch_types=[pltpu.VMEM_SHARED((N, LATENT), _f32),
                       pltpu.VMEM((_EPW,), jnp.int32),
                       pltpu.VMEM((_EPW,), jnp.int32),
                       pltpu.VMEM((_EPW, LATENT), _f32),
                       pltpu.VMEM((_EPW, LATENT), _f32)]
        + [pltpu.SemaphoreType.DMA] * 5,
    )


def _scat2(sa, ia, sb, ib, init):
    return _mk_scat2()(sa, ia, sb, ib, init)


def _eblk(i):
    return (i, 0)


def _full(i):
    return (0, 0)


def _espec(w=LATENT):
    return pl.BlockSpec((_EB, w), _eblk)


def _wspec(shape):
    return pl.BlockSpec(shape, _full)


def _enc_node_body(nodes, Wn0, bn0, Wn1, bn1, V0):
    a = jnp.maximum(_mm(nodes[...], Wn0[...]) + bn0[...], 0.0)
    V0[...], _ = _ln(_mm(a, Wn1[...]) + bn1[...])


def _run_enc_node(nodes, Wn0, bn0, Wn1, bn1):
    return pl.pallas_call(
        _enc_node_body,
        out_shape=jax.ShapeDtypeStruct((N, LATENT), _f32),
    )(nodes, Wn0, bn0, Wn1, bn1)


def _enc_edge_body(state, t2, w10, b10, W11, b11, w20, b20, W21, b21, Eh0):
    st = state[...]
    a1 = jnp.maximum(st * w10[...] + b10[...], 0.0)
    e1, _ = _ln(_mm(a1, W11[...]) + b11[...])
    a2 = jnp.maximum(st * w20[...] + b20[...], 0.0)
    e2, _ = _ln(_mm(a2, W21[...]) + b21[...])
    Eh0[...] = e1 + t2[...] * (e2 - e1)


def _run_enc_edge(state_col, t2_col, ew):
    return pl.pallas_call(
        _enc_edge_body,
        grid=(_NEB,),
        in_specs=[pl.BlockSpec((_EB, 1), _eblk), pl.BlockSpec((_EB, 1), _eblk)]
        + [_wspec(w.shape) for w in ew],
        out_specs=_espec(),
        out_shape=jax.ShapeDtypeStruct((E, LATENT), _f32),
    )(state_col, t2_col, *ew)


def _edge_fwd_body(Eh, Vs, Vr, P0, p0, P1, p1, Ehn, h_o, m_o, sig_o):
    c = jnp.concatenate([Eh[...], Vs[...], Vr[...]], axis=-1)
    h = jnp.maximum(_mm(c, P0[...]) + p0[...], 0.0)
    m, sig = _ln(_mm(h, P1[...]) + p1[...])
    Ehn[...] = Eh[...] + m
    h_o[...] = h
    m_o[...] = m
    sig_o[...] = sig


def _run_edge_fwd(Eh, Vs, Vr, pw):
    return pl.pallas_call(
        _edge_fwd_body,
        grid=(_NEB,),
        in_specs=[_espec(), _espec(), _espec()]
        + [_wspec(w.shape) for w in pw],
        out_specs=[_espec(), _espec(), _espec(), pl.BlockSpec((_EB, 1), _eblk)],
        out_shape=[jax.ShapeDtypeStruct((E, LATENT), _f32)] * 3
        + [jax.ShapeDtypeStruct((E, 1), _f32)],
    )(Eh, Vs, Vr, *pw)


def _node_fwd_body(V, aggp, Q0, q0, Q1, q1, Vn, k_o, u_o, sig_o):
    agg = aggp[0] + aggp[1]
    d = jnp.concatenate([V[...], agg], axis=-1)
    k = jnp.maximum(_mm(d, Q0[...]) + q0[...], 0.0)
    u, sig = _ln(_mm(k, Q1[...]) + q1[...])
    Vn[...] = V[...] + u
    k_o[...] = k
    u_o[...] = u
    sig_o[...] = sig


def _run_node_fwd(V, aggp, qw):
    return pl.pallas_call(
        _node_fwd_body,
        out_shape=[jax.ShapeDtypeStruct((N, LATENT), _f32)] * 3
        + [jax.ShapeDtypeStruct((N, 1), _f32)],
    )(V, aggp, *qw)


def _mid_body(Eh, Vs, Vr, t2, P0, p0, P1, p1,
              A0, a0, A1, a1, a2r, B0, b0, B1, b1, b2r,
              gE_o, gcs_o, gcr_o):
    c = jnp.concatenate([Eh[...], Vs[...], Vr[...]], axis=-1)
    h = jnp.maximum(_mm(c, P0[...]) + p0[...], 0.0)
    m, sig = _ln(_mm(h, P1[...]) + p1[...])
    Eh3 = Eh[...] + m
    t2v = t2[...]
    y11 = jnp.maximum(_mm(Eh3, A0[...]) + a0[...], 0.0)
    y12 = jnp.maximum(_mm(y11, A1[...]) + a1[...], 0.0)
    y21 = jnp.maximum(_mm(Eh3, B0[...]) + b0[...], 0.0)
    y22 = jnp.maximum(_mm(y21, B1[...]) + b1[...], 0.0)
    g12 = (1.0 - t2v) * a2r[...] * (y12 > 0)
    g11 = _mmT(g12, A1[...]) * (y11 > 0)
    gE = _mmT(g11, A0[...])
    g22 = t2v * b2r[...] * (y22 > 0)
    g21 = _mmT(g22, B1[...]) * (y21 > 0)
    gE = gE + _mmT(g21, B0[...])
    gz = _ln_bwd(gE, m, sig)
    gh = _mmT(gz, P1[...]) * (h > 0)
    gc = _mmT(gh, P0[...])
    gE_o[...] = gE + gc[:, :LATENT]
    gcs_o[...] = gc[:, LATENT:2 * LATENT]
    gcr_o[...] = gc[:, 2 * LATENT:]


def _run_mid(Eh, Vs, Vr, t2_col, pw, dw):
    return pl.pallas_call(
        _mid_body,
        grid=(_NEB,),
        in_specs=[_espec(), _espec(), _espec(), pl.BlockSpec((_EB, 1), _eblk)]
        + [_wspec(w.shape) for w in pw + dw],
        out_specs=[_espec(), _espec(), _espec()],
        out_shape=[jax.ShapeDtypeStruct((E, LATENT), _f32)] * 3,
    )(Eh, Vs, Vr, t2_col, *pw, *dw)


def _node_bwd_body(gVp, u, sigw, k, Q0, Q1, gVmid_o, gagg_o):
    gV = gVp[0] + gVp[1]
    gw = _ln_bwd(gV, u[...], sigw[...])
    gk = _mmT(gw, Q1[...]) * (k[...] > 0)
    gd = _mmT(gk, Q0[...])
    gVmid_o[...] = gV + gd[:, :LATENT]
    gagg_o[...] = gd[:, LATENT:]


def _run_node_bwd(gVp, u, sigw, k, Q0, Q1):
    return pl.pallas_call(
        _node_bwd_body,
        out_shape=[jax.ShapeDtypeStruct((N, LATENT), _f32)] * 2,
    )(gVp, u, sigw, k, Q0, Q1)


def _edge_bwd_body(gE_in, garr, m, sig, h, P0, P1, gE_o, gcs_o, gcr_o):
    gE = gE_in[...] + garr[...]
    gz = _ln_bwd(gE, m[...], sig[...])
    gh = _mmT(gz, P1[...]) * (h[...] > 0)
    gc = _mmT(gh, P0[...])
    gE_o[...] = gE + gc[:, :LATENT]
    gcs_o[...] = gc[:, LATENT:2 * LATENT]
    gcr_o[...] = gc[:, 2 * LATENT:]


def _run_edge_bwd(gE_in, garr, m, sig, h, P0, P1):
    return pl.pallas_call(
        _edge_bwd_body,
        grid=(_NEB,),
        in_specs=[_espec(), _espec(), _espec(), pl.BlockSpec((_EB, 1), _eblk),
                  _espec(), _wspec(P0.shape), _wspec(P1.shape)],
        out_specs=[_espec(), _espec(), _espec()],
        out_shape=[jax.ShapeDtypeStruct((E, LATENT), _f32)] * 3,
    )(gE_in, garr, m, sig, h, P0, P1)


def _edge_bwd0_enc_body(gE_in, garr, m, sig, h, P0, P1,
                        state, t2, w10, b10, W11, b11, w20, b20, W21, b21,
                        dH_o):
    gE = gE_in[...] + garr[...]
    gz = _ln_bwd(gE, m[...], sig[...])
    gh = _mmT(gz, P1[...]) * (h[...] > 0)
    gc = _mmT(gh, P0[...])
    gE0 = gE + gc[:, :LATENT]
    st = state[...]
    t2v = t2[...]
    a1 = jnp.maximum(st * w10[...] + b10[...], 0.0)
    e1, sig1 = _ln(_mm(a1, W11[...]) + b11[...])
    a2 = jnp.maximum(st * w20[...] + b20[...], 0.0)
    e2, sig2 = _ln(_mm(a2, W21[...]) + b21[...])
    gz1 = _ln_bwd((1.0 - t2v) * gE0, e1, sig1)
    ga1 = _mmT(gz1, W11[...]) * (a1 > 0)
    gs1 = jnp.sum(ga1 * w10[...], axis=-1, keepdims=True)
    gz2 = _ln_bwd(t2v * gE0, e2, sig2)
    ga2 = _mmT(gz2, W21[...]) * (a2 > 0)
    gs2 = jnp.sum(ga2 * w20[...], axis=-1, keepdims=True)
    dH_o[...] = gs1 + gs2


def _run_edge_bwd0_enc(gE_in, garr, m, sig, h, P0, P1, state_col, t2_col, ew):
    return pl.pallas_call(
        _edge_bwd0_enc_body,
        grid=(_NEB,),
        in_specs=[_espec(), _espec(), _espec(), pl.BlockSpec((_EB, 1), _eblk),
                  _espec(), _wspec(P0.shape), _wspec(P1.shape),
                  pl.BlockSpec((_EB, 1), _eblk), pl.BlockSpec((_EB, 1), _eblk)]
        + [_wspec(w.shape) for w in ew],
        out_specs=pl.BlockSpec((_EB, 1), _eblk),
        out_shape=jax.ShapeDtypeStruct((E, 1), _f32),
    )(gE_in, garr, m, sig, h, P0, P1, state_col, t2_col, *ew)


_JBLK = 512


def _jg_body(J_ref, g_ref, dHr_ref, dHc_ref, ctrl_ref, state_ref,
             out_ref, acc_ref):
    i = pl.program_id(0)

    @pl.when(i == 0)
    def _():
        acc_ref[...] = jnp.zeros_like(acc_ref)

    rows = lax.broadcasted_iota(jnp.int32, (_JBLK, E), 0) + i * _JBLK
    cols = lax.broadcasted_iota(jnp.int32, (_JBLK, E), 1)
    Jm = J_ref[...] * (cols >= rows).astype(_f32)
    y_rows = jnp.sum(Jm * dHr_ref[...], axis=1)
    cc = jnp.sum(Jm * dHc_ref[...], axis=0)
    acc_ref[0, :] = acc_ref[0, :] + cc
    gc = jnp.sum(g_ref[...] * ctrl_ref[...], axis=1)
    y_cols = acc_ref[0, pl.ds(i * _JBLK, _JBLK)]
    out_ref[:, 0] = state_ref[:, 0] + DT * (y_rows - y_cols + gc)


def _run_jg(J, g, dH_row, dH_col, ctrl_row, state_col):
    return pl.pallas_call(
        _jg_body,
        grid=(E // _JBLK,),
        in_specs=[
            pl.BlockSpec((_JBLK, E), _eblk),
            pl.BlockSpec((_JBLK, E), _eblk),
            pl.BlockSpec((1, E), _full),
            pl.BlockSpec((_JBLK, 1), _eblk),
            pl.BlockSpec((1, E), _full),
            pl.BlockSpec((_JBLK, 1), _eblk),
        ],
        out_specs=pl.BlockSpec((_JBLK, 1), _eblk),
        out_shape=jax.ShapeDtypeStruct((E, 1), _f32),
        scratch_shapes=[pltpu.VMEM((1, E), _f32)],
    )(J, g, dH_row, dH_col, ctrl_row, state_col)


def kernel(nodes, state, senders, receivers, type2_mask, control, J, g, params):
    state_col = state[:, None]
    send = senders.astype(jnp.int32)
    recv = receivers.astype(jnp.int32)
    t2_col = type2_mask[:, None].astype(_f32)

    def row(v):
        return v.reshape(1, -1)

    pe = params
    nw = (pe["enc_node"][0][0], row(pe["enc_node"][0][1]),
          pe["enc_node"][1][0], row(pe["enc_node"][1][1]))
    ew = (pe["enc_e1"][0][0], row(pe["enc_e1"][0][1]),
          pe["enc_e1"][1][0], row(pe["enc_e1"][1][1]),
          pe["enc_e2"][0][0], row(pe["enc_e2"][0][1]),
          pe["enc_e2"][1][0], row(pe["enc_e2"][1][1]))
    pw = (pe["proc_edge"][0][0], row(pe["proc_edge"][0][1]),
          pe["proc_edge"][1][0], row(pe["proc_edge"][1][1]))
    qw = (pe["proc_node"][0][0], row(pe["proc_node"][0][1]),
          pe["proc_node"][1][0], row(pe["proc_node"][1][1]))
    dw = (pe["dec_e1"][0][0], row(pe["dec_e1"][0][1]),
          pe["dec_e1"][1][0], row(pe["dec_e1"][1][1]),
          row(pe["dec_e1"][2][0][:, 0]),
          pe["dec_e2"][0][0], row(pe["dec_e2"][0][1]),
          pe["dec_e2"][1][0], row(pe["dec_e2"][1][1]),
          row(pe["dec_e2"][2][0][:, 0]))
    Q0, Q1 = qw[0], qw[2]
    P0, P1 = pw[0], pw[2]

    zinit = jnp.zeros((_NC, N, LATENT), _f32)

    V0 = _run_enc_node(nodes, *nw)
    Eh0 = _run_enc_edge(state_col, t2_col, ew)

    Vs0, Vr0 = _gather2(V0, send, recv)
    Eh1, h0, m0, sz0 = _run_edge_fwd(Eh0, Vs0, Vr0, pw)
    aggp0 = _scat1(Eh1, recv, zinit)
    V1, k0, u0, sw0 = _run_node_fwd(V0, aggp0, qw)

    Vs1, Vr1 = _gather2(V1, send, recv)
    Eh2, h1, m1, sz1 = _run_edge_fwd(Eh1, Vs1, Vr1, pw)
    aggp1 = _scat1(Eh2, recv, zinit)
    V2, k1, u1, sw1 = _run_node_fwd(V1, aggp1, qw)

    Vs2, Vr2 = _gather2(V2, send, recv)
    gE2, gcs2, gcr2 = _run_mid(Eh2, Vs2, Vr2, t2_col, pw, dw)

    gVp2 = _scat2(gcs2, send, gcr2, recv, zinit)
    gVmid1, gagg1 = _run_node_bwd(gVp2, u1, sw1, k1, Q0, Q1)
    garr1 = _gather1(gagg1, recv)
    gE1, gcs1, gcr1 = _run_edge_bwd(gE2, garr1, m1, sz1, h1, P0, P1)

    init1 = jnp.concatenate([gVmid1[None], jnp.zeros((1, N, LATENT), _f32)], 0)
    gVp1 = _scat2(gcs1, send, gcr1, recv, init1)
    _, gagg0 = _run_node_bwd(gVp1, u0, sw0, k0, Q0, Q1)
    garr0 = _gather1(gagg0, recv)
    dH_col = _run_edge_bwd0_enc(gE1, garr0, m0, sz0, h0, P0, P1,
                                state_col, t2_col, ew)

    dH_row = dH_col.reshape(1, E)
    next_col = _run_jg(J, g, dH_row, dH_col, control.reshape(1, E), state_col)
    return next_col[:, 0]
